# baseline (speedup 1.0000x reference)
import functools

import jax
import jax.numpy as jnp
from jax import lax
from jax.experimental import pallas as pl
from jax.experimental.pallas import tpu as pltpu
from jax.experimental.pallas import tpu_sc as plsc

_CHUNKS = 8
_C = 16


def _tc_body(emb_ref, out_ref, buf, rsem, wsem):
    batch = out_ref.shape[0]
    seq_len = out_ref.shape[1]
    rows = seq_len // _CHUNKS

    reads = []
    for i in range(_CHUNKS):
        c = pltpu.make_async_copy(
            emb_ref.at[pl.ds(i * rows, rows)],
            buf.at[pl.ds(i * rows, rows)],
            rsem.at[i],
        )
        c.start()
        reads.append(c)

    writes = []
    for i in range(_CHUNKS):
        reads[i].wait()
        for b in range(batch):
            c = pltpu.make_async_copy(
                buf.at[pl.ds(i * rows, rows)],
                out_ref.at[b, pl.ds(i * rows, rows)],
                wsem.at[i, b],
            )
            c.start()
            writes.append(c)

    for c in writes:
        c.wait()


def _tc_part(pos_embedding, batch, seq_len, d_model):
    return pl.pallas_call(
        _tc_body,
        in_specs=[pl.BlockSpec(memory_space=pl.ANY)],
        out_specs=pl.BlockSpec(memory_space=pl.ANY),
        out_shape=jax.ShapeDtypeStruct((batch, seq_len, d_model),
                                       pos_embedding.dtype),
        scratch_shapes=[
            pltpu.VMEM((seq_len, d_model), jnp.float32),
            pltpu.SemaphoreType.DMA((_CHUNKS,)),
            pltpu.SemaphoreType.DMA((_CHUNKS, batch)),
        ],
    )(pos_embedding)


def _sc_part(pos_embedding, batch, seq_len, d_model):
    info = plsc.get_sparse_core_info()
    nc, ns = info.num_cores, info.num_subcores
    nw = nc * ns
    rows_per_w = seq_len // nw
    k = rows_per_w // _C

    mesh = plsc.VectorSubcoreMesh(core_axis_name="c", subcore_axis_name="s")

    @functools.partial(
        pl.kernel,
        mesh=mesh,
        out_type=jax.ShapeDtypeStruct((batch, seq_len, d_model), jnp.float32),
        scratch_types=[
            pltpu.VMEM((2, _C, d_model), jnp.float32),
            pltpu.SemaphoreType.DMA((2,)),
            pltpu.SemaphoreType.DMA((2,)),
        ],
    )
    def k_sc(table_hbm, out_hbm, buf, rsem, wsem):
        wid = lax.axis_index("s") * nc + lax.axis_index("c")
        base = wid * rows_per_w

        writes = {}
        for j in range(k):
            s = j % 2
            r0 = base + j * _C
            if j >= 2:
                for c in writes.pop(j - 2):
                    c.wait()
            rd = pltpu.make_async_copy(
                table_hbm.at[pl.ds(r0, _C)], buf.at[s], rsem.at[s]
            )
            rd.start()
            rd.wait()
            ws = []
            for b in range(batch):
                wr = pltpu.make_async_copy(
                    buf.at[s], out_hbm.at[b, pl.ds(r0, _C)], wsem.at[s]
                )
                wr.start()
                ws.append(wr)
            writes[j] = ws
        for js in sorted(writes):
            for c in writes[js]:
                c.wait()

    return k_sc(pos_embedding)


def kernel(x, pos_embedding):
    batch, seq_len = x.shape
    max_len, d_model = pos_embedding.shape

    sc_batch = 1
    tc_batch = batch - sc_batch

    tc_out = _tc_part(pos_embedding, tc_batch, seq_len, d_model)
    sc_out = _sc_part(pos_embedding, sc_batch, seq_len, d_model)
    return jnp.concatenate([tc_out, sc_out], axis=0)


# speedup vs baseline: 3.2825x; 3.2825x over previous
import jax
import jax.numpy as jnp
from jax.experimental import pallas as pl
from jax.experimental.pallas import tpu as pltpu

_CHUNKS = 4


def _staged_body(emb_ref, out_ref, buf, rsem, wsem):
    batch = out_ref.shape[0]
    seq_len = out_ref.shape[1]
    rows = seq_len // _CHUNKS

    reads = []
    for i in range(_CHUNKS):
        c = pltpu.make_async_copy(
            emb_ref.at[pl.ds(i * rows, rows)],
            buf.at[pl.ds(i * rows, rows)],
            rsem.at[i],
        )
        c.start()
        reads.append(c)

    writes = []
    for i in range(_CHUNKS):
        reads[i].wait()
        for b in range(batch):
            c = pltpu.make_async_copy(
                buf.at[pl.ds(i * rows, rows)],
                out_ref.at[b, pl.ds(i * rows, rows)],
                wsem.at[i, b],
            )
            c.start()
            writes.append(c)

    for c in writes:
        c.wait()


def kernel(x, pos_embedding):
    batch, seq_len = x.shape
    max_len, d_model = pos_embedding.shape

    out = pl.pallas_call(
        _staged_body,
        in_specs=[pl.BlockSpec(memory_space=pl.ANY)],
        out_specs=pl.BlockSpec(memory_space=pl.ANY),
        out_shape=jax.ShapeDtypeStruct((batch, seq_len, d_model),
                                       pos_embedding.dtype),
        scratch_shapes=[
            pltpu.VMEM((seq_len, d_model), jnp.float32),
            pltpu.SemaphoreType.DMA((_CHUNKS,)),
            pltpu.SemaphoreType.DMA((_CHUNKS, 4)),
        ],
    )(pos_embedding)
    return out
